# baseline (speedup 1.0000x reference)
import functools

import jax
import jax.numpy as jnp
from jax import lax
from jax.experimental import pallas as pl
from jax.experimental.pallas import tpu as pltpu
from jax.experimental.pallas import tpu_sc as plsc

D = 64
T = 200
B = 4096
NW = 32
BPW = B // NW

_mesh = plsc.VectorSubcoreMesh(core_axis_name="c", subcore_axis_name="s")


@functools.partial(
    pl.kernel,
    out_type=jax.ShapeDtypeStruct((T, D // 8, B // 128, 8, 128), jnp.float32),
    mesh=_mesh,
    scratch_types=[
        pltpu.VMEM((T, BPW), jnp.int32),
        pltpu.VMEM((T, D), jnp.float32),
        [pltpu.VMEM((BPW, D), jnp.float32) for _ in range(8)],
        [pltpu.VMEM((D // 8, 8, BPW + 1), jnp.float32) for _ in range(2)],
        [pltpu.SemaphoreType.DMA for _ in range(8)],
        [pltpu.SemaphoreType.DMA for _ in range(2)],
    ],
    compiler_params=pltpu.CompilerParams(
        use_tc_tiling_on_sc=False, needs_layout_passes=False
    ),
)
def _emb(st_hbm, word_hbm, pos_hbm, out_hbm, idx_t, pos_v, grows,
         tbuf, gsem, osem):
    wid = lax.axis_index("s") * 2 + lax.axis_index("c")
    pltpu.sync_copy(st_hbm.at[:, pl.ds(wid * BPW, BPW)], idx_t)
    pltpu.sync_copy(pos_hbm, pos_v)

    lane = lax.iota(jnp.int32, 16)
    dvs = [lane + c * 16 for c in range(D // 16)]
    dgvs = [dv >> 3 for dv in dvs]
    divs = [dv & 7 for dv in dvs]

    def fire_gather(t, k):
        pltpu.async_copy(word_hbm.at[idx_t.at[t]], grows[k], gsem[k])

    def gdrain(k):
        pltpu.make_async_copy(word_hbm.at[pl.ds(0, BPW)], grows[k], gsem[k]).wait()

    def fire_out(t, k):
        pltpu.async_copy(
            tbuf[k].at[:, :, pl.ds(0, BPW)],
            out_hbm.at[t, :, wid],
            osem[k],
        )

    def odrain(k):
        pltpu.make_async_copy(
            out_hbm.at[0, :, 0],
            tbuf[k].at[:, :, pl.ds(0, BPW)],
            osem[k],
        ).wait()

    for w in range(8):
        fire_gather(w, w)

    def main_body(jj, carry):
        for k in range(8):
            t = jj * 8 + k
            tb = k % 2
            gdrain(k)

            if k < 2:
                @pl.when(jj > 0)
                def _():
                    odrain(tb)
            else:
                odrain(tb)

            ps = [pos_v[t, pl.ds(c * 16, 16)] for c in range(D // 16)]

            @plsc.parallel_loop(0, BPW, unroll=4)
            def row_body(b):
                bv = jnp.full((16,), b, jnp.int32)
                for c in range(D // 16):
                    v = grows[k][b, pl.ds(c * 16, 16)] + ps[c]
                    plsc.store_scatter(tbuf[tb], [dgvs[c], divs[c], bv], v)

            fire_out(t, tb)

            @pl.when(jj < T // 8 - 1)
            def _():
                fire_gather(t + 8, k)
        return carry

    lax.fori_loop(0, T // 8, main_body, 0)
    odrain(0)
    odrain(1)


def kernel(sentence, word_emb, pos_emb):
    pos_slice = lax.slice(pos_emb, (1, 0), (T + 1, D))
    out5 = _emb(sentence, word_emb, pos_slice)
    return out5.transpose(2, 4, 0, 1, 3).reshape(B, T, D)


# speedup vs baseline: 1.0058x; 1.0058x over previous
import functools

import jax
import jax.numpy as jnp
from jax import lax
from jax.experimental import pallas as pl
from jax.experimental.pallas import tpu as pltpu
from jax.experimental.pallas import tpu_sc as plsc

D = 64
T = 200
B = 4096
NW = 32
BPW = B // NW

_mesh = plsc.VectorSubcoreMesh(core_axis_name="c", subcore_axis_name="s")


@functools.partial(
    pl.kernel,
    out_type=jax.ShapeDtypeStruct((T, D // 8, B // 128, 8, 128), jnp.float32),
    mesh=_mesh,
    scratch_types=[
        pltpu.VMEM((T, BPW), jnp.int32),
        pltpu.VMEM((T, D), jnp.float32),
        [pltpu.VMEM((BPW, D), jnp.float32) for _ in range(4)],
        [pltpu.VMEM((D // 8, 8, BPW + 1), jnp.float32) for _ in range(4)],
        [pltpu.SemaphoreType.DMA for _ in range(4)],
        [pltpu.SemaphoreType.DMA for _ in range(4)],
    ],
    compiler_params=pltpu.CompilerParams(
        use_tc_tiling_on_sc=False, needs_layout_passes=False
    ),
)
def _emb(st_hbm, word_hbm, pos_hbm, out_hbm, idx_t, pos_v, grows,
         tbuf, gsem, osem):
    wid = lax.axis_index("s") * 2 + lax.axis_index("c")
    pltpu.sync_copy(st_hbm.at[:, pl.ds(wid * BPW, BPW)], idx_t)
    pltpu.sync_copy(pos_hbm, pos_v)

    lane = lax.iota(jnp.int32, 16)
    dvs = [lane + c * 16 for c in range(D // 16)]
    dgvs = [dv >> 3 for dv in dvs]
    divs = [dv & 7 for dv in dvs]

    def fire_gather(t, k):
        pltpu.async_copy(word_hbm.at[idx_t.at[t]], grows[k], gsem[k])

    def gdrain(k):
        pltpu.make_async_copy(word_hbm.at[pl.ds(0, BPW)], grows[k], gsem[k]).wait()

    def fire_out(t, k):
        pltpu.async_copy(
            tbuf[k].at[:, :, pl.ds(0, BPW)],
            out_hbm.at[t, :, wid],
            osem[k],
        )

    def odrain(k):
        pltpu.make_async_copy(
            out_hbm.at[0, :, 0],
            tbuf[k].at[:, :, pl.ds(0, BPW)],
            osem[k],
        ).wait()

    for w in range(4):
        fire_gather(w, w)

    def main_body(jj, carry):
        for k in range(4):
            t = jj * 4 + k
            tb = k
            gdrain(k)

            @pl.when(jj > 0)
            def _():
                odrain(tb)

            ps = [pos_v[t, pl.ds(c * 16, 16)] for c in range(D // 16)]

            @plsc.parallel_loop(0, BPW, unroll=4)
            def row_body(b):
                bv = jnp.full((16,), b, jnp.int32)
                for c in range(D // 16):
                    v = grows[k][b, pl.ds(c * 16, 16)] + ps[c]
                    plsc.store_scatter(tbuf[tb], [dgvs[c], divs[c], bv], v)

            fire_out(t, tb)

            @pl.when(jj < T // 4 - 1)
            def _():
                fire_gather(t + 4, k)
        return carry

    lax.fori_loop(0, T // 4, main_body, 0)
    for w in range(4):
        odrain(w)


def kernel(sentence, word_emb, pos_emb):
    pos_slice = lax.slice(pos_emb, (1, 0), (T + 1, D))
    out5 = _emb(sentence, word_emb, pos_slice)
    return out5.transpose(2, 4, 0, 1, 3).reshape(B, T, D)
